# Initial kernel scaffold; baseline (speedup 1.0000x reference)
import jax
import jax.numpy as jnp
from jax.experimental import pallas as pl


def kernel(cls_p4, reg_p4, cls_p5, reg_p5, t4, t5):
    raise NotImplementedError("write your pallas kernel here")


# speedup vs baseline: 1.6073x; 1.6073x over previous
import jax
import jax.numpy as jnp
from jax.experimental import pallas as pl
from jax.experimental.pallas import tpu as pltpu

_C = 81
_NC = 80
_ALPHA = 0.25
_BBOX_W, _OBJ_W, _CLS_W = 2.0, 1.0, 0.5


def _softplus(x):
    return jnp.maximum(x, 0.0) + jnp.log1p(jnp.exp(-jnp.abs(x)))


def _smooth_l1(pred, tgt):
    d = pred - tgt
    ad = jnp.abs(d)
    return jnp.where(ad < 1.0, 0.5 * d * d, ad - 0.5)


def _scale_terms(cls_b, reg_b, t_b, hw, w):
    h = hw // w
    tx = t_b[1:2, :] * w
    ty = t_b[2:3, :] * h
    tw = t_b[3:4, :] * w
    th = t_b[4:5, :] * h
    cls_ids = t_b[0:1, :].astype(jnp.int32)
    gx = jnp.clip(tx, 0.0, w - 1.0).astype(jnp.int32)
    gy = jnp.clip(ty, 0.0, h - 1.0).astype(jnp.int32)
    idx = gy * w + gx

    tcount = t_b.shape[1]
    iota_hw = jax.lax.broadcasted_iota(jnp.int32, (hw, tcount), 0)
    onehot = (iota_hw == idx).astype(jnp.float32)

    dims = (((1,), (0,)), ((), ()))
    cls_at = jax.lax.dot_general(cls_b, onehot, dims,
                                 preferred_element_type=jnp.float32)
    reg_at = jax.lax.dot_general(reg_b, onehot, dims,
                                 preferred_element_type=jnp.float32)

    dx = 1.0 / (1.0 + jnp.exp(-reg_at[0:1, :]))
    dy = 1.0 / (1.0 + jnp.exp(-reg_at[1:2, :]))
    dw = jnp.exp(jnp.clip(reg_at[2:3, :], -4.0, 4.0))
    dh = jnp.exp(jnp.clip(reg_at[3:4, :], -4.0, 4.0))
    px = gx.astype(jnp.float32) + dx
    py = gy.astype(jnp.float32) + dy
    sl = (_smooth_l1(px - dw * 0.5, tx - tw * 0.5)
          + _smooth_l1(py - dh * 0.5, ty - th * 0.5)
          + _smooth_l1(px + dw * 0.5, tx + tw * 0.5)
          + _smooth_l1(py + dh * 0.5, ty + th * 0.5)) * 0.25
    lb = jnp.sum(sl)

    obj_logit = cls_at[0:1, :]
    lo_pos = jnp.sum(_softplus(-obj_logit))

    logits = cls_at[1:, :]
    iota_c = jax.lax.broadcasted_iota(jnp.int32, (_NC, tcount), 0)
    oh = (iota_c == cls_ids).astype(jnp.float32)
    bce = _softplus(logits) - logits * oh
    p = 1.0 / (1.0 + jnp.exp(-logits))
    pt = p * oh + (1.0 - p) * (1.0 - oh)
    one_m_pt = 1.0 - pt
    focal = _ALPHA * one_m_pt * one_m_pt * bce
    lc = jnp.sum(focal) * (1.0 / _NC)

    hit = jnp.max(onehot, axis=1, keepdims=True)
    sp0 = _softplus(cls_b[0:1, :])
    hit_sp = jax.lax.dot_general(sp0, hit, dims,
                                 preferred_element_type=jnp.float32)
    bg_sum = jnp.sum(sp0) - hit_sp[0, 0]
    bg_cnt = hw - jnp.sum(hit)
    return lb, lo_pos, lc, bg_sum, bg_cnt


def _body(cls4_ref, reg4_ref, cls5_ref, reg5_ref, t4_ref, t5_ref,
          out_ref, acc_ref):
    i = pl.program_id(0)
    nb = pl.num_programs(0)

    @pl.when(i == 0)
    def _init():
        for k in range(8):
            acc_ref[k] = 0.0

    lb4, lo4, lc4, bs4, bc4 = _scale_terms(
        cls4_ref[0], reg4_ref[0], t4_ref[0], 1024, 32)
    lb5, lo5, lc5, bs5, bc5 = _scale_terms(
        cls5_ref[0], reg5_ref[0], t5_ref[0], 256, 16)

    acc_ref[0] = acc_ref[0] + lb4 + lb5
    acc_ref[1] = acc_ref[1] + lo4 + lo5
    acc_ref[2] = acc_ref[2] + lc4 + lc5
    acc_ref[3] = acc_ref[3] + bs4
    acc_ref[4] = acc_ref[4] + bc4
    acc_ref[5] = acc_ref[5] + bs5
    acc_ref[6] = acc_ref[6] + bc5

    @pl.when(i == nb - 1)
    def _fin():
        n = 256.0
        lb = acc_ref[0] / n
        lo = (acc_ref[1]
              + 0.05 * acc_ref[3] / acc_ref[4]
              + 0.05 * acc_ref[5] / acc_ref[6]) / n
        lc = acc_ref[2] / n
        out_ref[0, 0] = _BBOX_W * lb + _OBJ_W * lo + _CLS_W * lc


def kernel(cls_p4, reg_p4, cls_p5, reg_p5, t4, t5):
    b = cls_p4.shape[0]
    cls4r = cls_p4.reshape(b, _C, 32 * 32)
    reg4r = reg_p4.reshape(b, 4, 32 * 32)
    cls5r = cls_p5.reshape(b, _C, 16 * 16)
    reg5r = reg_p5.reshape(b, 4, 16 * 16)
    t4t = jnp.swapaxes(t4, 1, 2)
    t5t = jnp.swapaxes(t5, 1, 2)

    out = pl.pallas_call(
        _body,
        grid=(b,),
        in_specs=[
            pl.BlockSpec((1, _C, 1024), lambda i: (i, 0, 0)),
            pl.BlockSpec((1, 4, 1024), lambda i: (i, 0, 0)),
            pl.BlockSpec((1, _C, 256), lambda i: (i, 0, 0)),
            pl.BlockSpec((1, 4, 256), lambda i: (i, 0, 0)),
            pl.BlockSpec((1, 5, 8), lambda i: (i, 0, 0)),
            pl.BlockSpec((1, 5, 8), lambda i: (i, 0, 0)),
        ],
        out_specs=pl.BlockSpec(memory_space=pltpu.SMEM),
        out_shape=jax.ShapeDtypeStruct((1, 1), jnp.float32),
        scratch_shapes=[pltpu.SMEM((8,), jnp.float32)],
    )(cls4r, reg4r, cls5r, reg5r, t4t, t5t)
    return out.reshape(())
